# Initial kernel scaffold; baseline (speedup 1.0000x reference)
import jax
import jax.numpy as jnp
from jax.experimental import pallas as pl


def kernel(gate_features, W1, b1, W2, b2):
    raise NotImplementedError("write your pallas kernel here")


# speedup vs baseline: 1.1278x; 1.1278x over previous
import jax
import jax.numpy as jnp
from jax.experimental import pallas as pl
from jax.experimental.pallas import tpu as pltpu

TOKENS = 32768
GATE_DIM = 2048
HIDDEN_DIM = 256
NUM_EXPERTS = 64
TBLK = 512


def _router_body(x_ref, w1_ref, b1_ref, w2_ref, b2_ref,
                 probs_ref, i1_ref, i2_ref, p1_ref, p2_ref):
    x = x_ref[...]
    h = jax.lax.dot_general(x, w1_ref[...], (((1,), (0,)), ((), ())),
                            preferred_element_type=jnp.float32)
    h = jnp.maximum(h + b1_ref[...], 0.0)
    logits = jax.lax.dot_general(h, w2_ref[...], (((1,), (0,)), ((), ())),
                                 preferred_element_type=jnp.float32)
    logits = logits + b2_ref[...]

    m1 = jnp.max(logits, axis=1, keepdims=True)
    e = jnp.exp(logits - m1)
    s = jnp.sum(e, axis=1, keepdims=True)
    inv_s = 1.0 / s
    probs_ref[...] = e * inv_s

    iota = jax.lax.broadcasted_iota(jnp.int32, (TBLK, NUM_EXPERTS), 1)
    big = jnp.int32(NUM_EXPERTS)
    i1 = jnp.min(jnp.where(logits == m1, iota, big), axis=1, keepdims=True)
    l2 = jnp.where(iota == i1, -jnp.inf, logits)
    m2 = jnp.max(l2, axis=1, keepdims=True)
    i2 = jnp.min(jnp.where(l2 == m2, iota, big), axis=1, keepdims=True)

    i1_ref[...] = i1
    i2_ref[...] = i2
    p1_ref[...] = inv_s
    p2_ref[...] = jnp.exp(m2 - m1) * inv_s


def kernel(gate_features, W1, b1, W2, b2):
    nblk = TOKENS // TBLK
    probs, i1, i2, p1, p2 = pl.pallas_call(
        _router_body,
        grid=(nblk,),
        in_specs=[
            pl.BlockSpec((TBLK, GATE_DIM), lambda i: (i, 0)),
            pl.BlockSpec((GATE_DIM, HIDDEN_DIM), lambda i: (0, 0)),
            pl.BlockSpec((HIDDEN_DIM,), lambda i: (0,)),
            pl.BlockSpec((HIDDEN_DIM, NUM_EXPERTS), lambda i: (0, 0)),
            pl.BlockSpec((NUM_EXPERTS,), lambda i: (0,)),
        ],
        out_specs=[
            pl.BlockSpec((TBLK, NUM_EXPERTS), lambda i: (i, 0)),
            pl.BlockSpec((TBLK, 1), lambda i: (i, 0)),
            pl.BlockSpec((TBLK, 1), lambda i: (i, 0)),
            pl.BlockSpec((TBLK, 1), lambda i: (i, 0)),
            pl.BlockSpec((TBLK, 1), lambda i: (i, 0)),
        ],
        out_shape=[
            jax.ShapeDtypeStruct((TOKENS, NUM_EXPERTS), jnp.float32),
            jax.ShapeDtypeStruct((TOKENS, 1), jnp.int32),
            jax.ShapeDtypeStruct((TOKENS, 1), jnp.int32),
            jax.ShapeDtypeStruct((TOKENS, 1), jnp.float32),
            jax.ShapeDtypeStruct((TOKENS, 1), jnp.float32),
        ],
    )(gate_features, W1, b1, W2, b2)

    assignments = i1[:, 0]
    topk_idx = jnp.concatenate([i1, i2], axis=1)
    topk_probs = jnp.concatenate([p1, p2], axis=1)
    return (assignments, probs, topk_idx, topk_probs)


# speedup vs baseline: 1.2697x; 1.1257x over previous
import jax
import jax.numpy as jnp
from jax.experimental import pallas as pl
from jax.experimental.pallas import tpu as pltpu

TOKENS = 32768
GATE_DIM = 2048
HIDDEN_DIM = 256
NUM_EXPERTS = 64
TBLK = 1024


def _router_body(x_ref, w1_ref, b1_ref, w2_ref, b2_ref,
                 probs_ref, i1_ref, i2_ref, p1_ref, p2_ref):
    x = x_ref[...]
    h = jax.lax.dot_general(x, w1_ref[...], (((1,), (0,)), ((), ())),
                            preferred_element_type=jnp.float32)
    h = jnp.maximum(h + b1_ref[...], 0.0)
    logits = jax.lax.dot_general(h, w2_ref[...], (((1,), (0,)), ((), ())),
                                 preferred_element_type=jnp.float32)
    logits = logits + b2_ref[...]

    m1 = jnp.max(logits, axis=1, keepdims=True)
    e = jnp.exp(logits - m1)
    s = jnp.sum(e, axis=1, keepdims=True)
    inv_s = 1.0 / s
    probs_ref[...] = e * inv_s

    iota = jax.lax.broadcasted_iota(jnp.int32, (TBLK, NUM_EXPERTS), 1)
    big = jnp.int32(NUM_EXPERTS)
    i1 = jnp.min(jnp.where(logits == m1, iota, big), axis=1, keepdims=True)
    l2 = jnp.where(iota == i1, -jnp.inf, logits)
    m2 = jnp.max(l2, axis=1, keepdims=True)
    i2 = jnp.min(jnp.where(l2 == m2, iota, big), axis=1, keepdims=True)

    i1_ref[...] = i1
    i2_ref[...] = i2
    p1_ref[...] = inv_s
    p2_ref[...] = jnp.exp(m2 - m1) * inv_s


def kernel(gate_features, W1, b1, W2, b2):
    nblk = TOKENS // TBLK
    probs, i1, i2, p1, p2 = pl.pallas_call(
        _router_body,
        grid=(nblk,),
        in_specs=[
            pl.BlockSpec((TBLK, GATE_DIM), lambda i: (i, 0)),
            pl.BlockSpec((GATE_DIM, HIDDEN_DIM), lambda i: (0, 0)),
            pl.BlockSpec((HIDDEN_DIM,), lambda i: (0,)),
            pl.BlockSpec((HIDDEN_DIM, NUM_EXPERTS), lambda i: (0, 0)),
            pl.BlockSpec((NUM_EXPERTS,), lambda i: (0,)),
        ],
        out_specs=[
            pl.BlockSpec((TBLK, NUM_EXPERTS), lambda i: (i, 0)),
            pl.BlockSpec((TBLK, 1), lambda i: (i, 0)),
            pl.BlockSpec((TBLK, 1), lambda i: (i, 0)),
            pl.BlockSpec((TBLK, 1), lambda i: (i, 0)),
            pl.BlockSpec((TBLK, 1), lambda i: (i, 0)),
        ],
        out_shape=[
            jax.ShapeDtypeStruct((TOKENS, NUM_EXPERTS), jnp.float32),
            jax.ShapeDtypeStruct((TOKENS, 1), jnp.int32),
            jax.ShapeDtypeStruct((TOKENS, 1), jnp.int32),
            jax.ShapeDtypeStruct((TOKENS, 1), jnp.float32),
            jax.ShapeDtypeStruct((TOKENS, 1), jnp.float32),
        ],
    )(gate_features, W1, b1, W2, b2)

    assignments = i1[:, 0]
    topk_idx = jnp.concatenate([i1, i2], axis=1)
    topk_probs = jnp.concatenate([p1, p2], axis=1)
    return (assignments, probs, topk_idx, topk_probs)


# speedup vs baseline: 1.3268x; 1.0450x over previous
import jax
import jax.numpy as jnp
from jax.experimental import pallas as pl
from jax.experimental.pallas import tpu as pltpu

TOKENS = 32768
GATE_DIM = 2048
HIDDEN_DIM = 256
NUM_EXPERTS = 64
TBLK = 2048


def _router_body(x_ref, w1_ref, b1_ref, w2_ref, b2_ref,
                 probs_ref, i1_ref, i2_ref, p1_ref, p2_ref):
    x = x_ref[...]
    h = jax.lax.dot_general(x, w1_ref[...], (((1,), (0,)), ((), ())),
                            preferred_element_type=jnp.float32)
    h = jnp.maximum(h + b1_ref[...], 0.0)
    logits = jax.lax.dot_general(h, w2_ref[...], (((1,), (0,)), ((), ())),
                                 preferred_element_type=jnp.float32)
    logits = logits + b2_ref[...]

    m1 = jnp.max(logits, axis=1, keepdims=True)
    e = jnp.exp(logits - m1)
    s = jnp.sum(e, axis=1, keepdims=True)
    inv_s = 1.0 / s
    probs_ref[...] = e * inv_s

    iota = jax.lax.broadcasted_iota(jnp.int32, (TBLK, NUM_EXPERTS), 1)
    big = jnp.int32(NUM_EXPERTS)
    i1 = jnp.min(jnp.where(logits == m1, iota, big), axis=1, keepdims=True)
    l2 = jnp.where(iota == i1, -jnp.inf, logits)
    m2 = jnp.max(l2, axis=1, keepdims=True)
    i2 = jnp.min(jnp.where(l2 == m2, iota, big), axis=1, keepdims=True)

    i1_ref[...] = i1
    i2_ref[...] = i2
    p1_ref[...] = inv_s
    p2_ref[...] = jnp.exp(m2 - m1) * inv_s


def kernel(gate_features, W1, b1, W2, b2):
    nblk = TOKENS // TBLK
    probs, i1, i2, p1, p2 = pl.pallas_call(
        _router_body,
        grid=(nblk,),
        in_specs=[
            pl.BlockSpec((TBLK, GATE_DIM), lambda i: (i, 0)),
            pl.BlockSpec((GATE_DIM, HIDDEN_DIM), lambda i: (0, 0)),
            pl.BlockSpec((HIDDEN_DIM,), lambda i: (0,)),
            pl.BlockSpec((HIDDEN_DIM, NUM_EXPERTS), lambda i: (0, 0)),
            pl.BlockSpec((NUM_EXPERTS,), lambda i: (0,)),
        ],
        out_specs=[
            pl.BlockSpec((TBLK, NUM_EXPERTS), lambda i: (i, 0)),
            pl.BlockSpec((TBLK, 1), lambda i: (i, 0)),
            pl.BlockSpec((TBLK, 1), lambda i: (i, 0)),
            pl.BlockSpec((TBLK, 1), lambda i: (i, 0)),
            pl.BlockSpec((TBLK, 1), lambda i: (i, 0)),
        ],
        out_shape=[
            jax.ShapeDtypeStruct((TOKENS, NUM_EXPERTS), jnp.float32),
            jax.ShapeDtypeStruct((TOKENS, 1), jnp.int32),
            jax.ShapeDtypeStruct((TOKENS, 1), jnp.int32),
            jax.ShapeDtypeStruct((TOKENS, 1), jnp.float32),
            jax.ShapeDtypeStruct((TOKENS, 1), jnp.float32),
        ],
    )(gate_features, W1, b1, W2, b2)

    assignments = i1[:, 0]
    topk_idx = jnp.concatenate([i1, i2], axis=1)
    topk_probs = jnp.concatenate([p1, p2], axis=1)
    return (assignments, probs, topk_idx, topk_probs)
